# baseline (speedup 1.0000x reference)
import functools

import jax
import jax.numpy as jnp
from jax import lax
from jax.experimental import pallas as pl
from jax.experimental.pallas import tpu as pltpu
from jax.experimental.pallas import tpu_sc as plsc

N = 10000
E = 320000
D = 128
G = 64

NC = 2
NS = 16
NW = NC * NS

NPAD = 10240
TRASH = N
CH = 80
B = 128
EPAD = NW * CH * B

CHT = 160
CH0 = 128

BN = 2048
NBLK = NPAD // BN


def _make_deg_kernel():
    mesh = plsc.VectorSubcoreMesh(core_axis_name="c", subcore_axis_name="s")

    @functools.partial(
        pl.kernel,
        out_type=jax.ShapeDtypeStruct((NC, NPAD), jnp.float32),
        mesh=mesh,
        scratch_types=[
            pltpu.VMEM((CH, B), jnp.int32),
            pltpu.VMEM((B,), jnp.float32),
            pltpu.VMEM((640,), jnp.float32),
            pltpu.VMEM_SHARED((NPAD,), jnp.float32),
        ],
    )
    def deg_kernel(dst_hbm, degp_hbm, didx, ones_v, zb, degS):
        c = lax.axis_index("c")
        s = lax.axis_index("s")
        wid = s * NC + c

        def zb_body(i, carry):
            zb[pl.ds(i * 16, 16)] = jnp.zeros((16,), jnp.float32)
            return carry
        lax.fori_loop(0, 640 // 16, zb_body, 0)
        for k in range(B // 16):
            ones_v[pl.ds(k * 16, 16)] = jnp.ones((16,), jnp.float32)

        pltpu.sync_copy(zb, degS.at[pl.ds(s * 640, 640)])
        plsc.subcore_barrier()

        pltpu.sync_copy(dst_hbm.at[wid], didx)

        def body(j, carry):
            pltpu.sync_copy(ones_v, degS.at[didx.at[j]], add=True)
            return carry
        lax.fori_loop(0, CH, body, 0)

        plsc.subcore_barrier()
        pltpu.sync_copy(degS.at[pl.ds(s * 640, 640)],
                        degp_hbm.at[c, pl.ds(s * 640, 640)])

    return deg_kernel


def _make_scatter_kernel():
    mesh = plsc.VectorSubcoreMesh(core_axis_name="c", subcore_axis_name="s")

    @functools.partial(
        pl.kernel,
        out_type=jax.ShapeDtypeStruct((NC, NPAD, D), jnp.float32),
        mesh=mesh,
        scratch_types=[
            pltpu.VMEM((4, 2, B), jnp.int32),
            pltpu.VMEM((2, B, D), jnp.float32),
            pltpu.VMEM_SHARED((NPAD, D), jnp.float32),
            pltpu.SemaphoreType.DMA,
            pltpu.SemaphoreType.DMA,
            pltpu.SemaphoreType.DMA,
            pltpu.SemaphoreType.DMA,
            pltpu.SemaphoreType.DMA,
            pltpu.SemaphoreType.DMA,
        ],
    )
    def scatter_kernel(g_hbm, idx_hbm, out_hbm, ibuf, rbuf, accS,
                       is0, is1, is2, is3, gs0, gs1):
        c = lax.axis_index("c")
        s = lax.axis_index("s")
        isems = (is0, is1, is2, is3)
        gsems = (gs0, gs1)

        def zr(r, carry):
            for k in range(D // 16):
                rbuf[0, r, pl.ds(k * 16, 16)] = jnp.zeros((16,), jnp.float32)
            return carry
        lax.fori_loop(0, B, zr, 0)
        for k in range(640 // B):
            pltpu.sync_copy(rbuf.at[0], accS.at[pl.ds(s * 640 + k * B, B)])
        plsc.subcore_barrier()

        def sweep(base, nch):
            pltpu.async_copy(idx_hbm.at[s, base], ibuf.at[0], isems[0])
            pltpu.async_copy(idx_hbm.at[s, base + 1], ibuf.at[1], isems[1])
            pltpu.make_async_copy(idx_hbm.at[s, base], ibuf.at[0],
                                  isems[0]).wait()
            pltpu.async_copy(g_hbm.at[ibuf.at[0, 0]], rbuf.at[0], gsems[0])

            def body(q, carry):
                for k in range(4):
                    l = 4 * q + k
                    j = base + l
                    k1 = (k + 1) % 4
                    k2 = (k + 2) % 4

                    @pl.when(l + 2 < nch)
                    def _():
                        pltpu.async_copy(idx_hbm.at[s, j + 2], ibuf.at[k2],
                                         isems[k2])

                    @pl.when(l + 1 < nch)
                    def _():
                        pltpu.make_async_copy(idx_hbm.at[s, j + 1],
                                              ibuf.at[k1], isems[k1]).wait()
                        pltpu.async_copy(g_hbm.at[ibuf.at[k1, 0]],
                                         rbuf.at[(k + 1) % 2],
                                         gsems[(k + 1) % 2])

                    pltpu.make_async_copy(g_hbm.at[ibuf.at[k, 0]],
                                          rbuf.at[k % 2], gsems[k % 2]).wait()
                    pltpu.sync_copy(rbuf.at[k % 2], accS.at[ibuf.at[k, 1]],
                                    add=True)
                return carry
            lax.fori_loop(0, nch // 4, body, 0)

        @pl.when(c == 0)
        def _():
            sweep(0, CH0)

        @pl.when(c == 1)
        def _():
            sweep(CH0, CHT - CH0)

        plsc.subcore_barrier()
        for k in range(640 // B):
            pltpu.sync_copy(accS.at[pl.ds(s * 640 + k * B, B)],
                            out_hbm.at[c, pl.ds(s * 640 + k * B, B)])

    return scatter_kernel


def _tc1_body(x_ref, w_ref, degp_ref, g_ref):
    deg = degp_ref[0] + degp_ref[1] + 1.0
    dis = lax.rsqrt(deg)
    h = jnp.dot(x_ref[...], w_ref[...], preferred_element_type=jnp.float32)
    g_ref[...] = h * dis


def _tc2_body(accp_ref, g1_ref, degp_ref, w_ref, b_ref, g2_ref):
    deg = degp_ref[0] + degp_ref[1] + 1.0
    dis = lax.rsqrt(deg)
    ssum = accp_ref[0] + accp_ref[1] + g1_ref[...]
    h1 = jnp.maximum(ssum * dis + b_ref[...], 0.0)
    g2_ref[...] = jnp.dot(h1, w_ref[...],
                          preferred_element_type=jnp.float32) * dis


def _tc3_body(accp_ref, g2_ref, degp_ref, b_ref, isn_ref, batch_ref, out_ref):
    i = pl.program_id(0)
    deg = degp_ref[0] + degp_ref[1] + 1.0
    dis = lax.rsqrt(deg)
    h2 = (accp_ref[0] + accp_ref[1] + g2_ref[...]) * dis + b_ref[...]
    h2 = h2 * (0.5 * isn_ref[...])
    onehot = (lax.broadcasted_iota(jnp.int32, (BN, G), 1)
              == batch_ref[...]).astype(jnp.float32)
    contrib = lax.dot_general(onehot, h2, (((0,), (0,)), ((), ())),
                              preferred_element_type=jnp.float32)

    @pl.when(i == 0)
    def _():
        out_ref[...] = contrib

    @pl.when(i != 0)
    def _():
        out_ref[...] = out_ref[...] + contrib


def kernel(x, edge_index, is_neighbor, batch, W1, b1, W2, b2):
    src = edge_index[0]
    dst = edge_index[1]
    npad_e = EPAD - E
    src_pad = jnp.concatenate([src, jnp.zeros((npad_e,), jnp.int32)])
    dst_pad = jnp.concatenate([dst, jnp.full((npad_e,), TRASH, jnp.int32)])
    dst_r = dst_pad.reshape(NW, CH, B)
    idx_both = jnp.stack(
        [src_pad.reshape(NS, CHT, B), dst_pad.reshape(NS, CHT, B)], axis=2)

    x_p = jnp.pad(x, ((0, NPAD - N), (0, 0)))
    isn_p = jnp.pad(is_neighbor, (0, NPAD - N)).reshape(NPAD, 1)
    batch_p = jnp.pad(batch, (0, NPAD - N),
                      constant_values=G).reshape(NPAD, 1)
    b1r = b1.reshape(1, D)
    b2r = b2.reshape(1, D)

    deg_kernel = _make_deg_kernel()
    scatter_kernel = _make_scatter_kernel()

    degp = deg_kernel(dst_r)
    degp3 = degp.reshape(NC, NPAD, 1)

    blk = lambda i: (i, 0)
    g1 = pl.pallas_call(
        _tc1_body,
        grid=(NBLK,),
        in_specs=[
            pl.BlockSpec((BN, D), blk),
            pl.BlockSpec((D, D), lambda i: (0, 0)),
            pl.BlockSpec((NC, BN, 1), lambda i: (0, i, 0)),
        ],
        out_specs=pl.BlockSpec((BN, D), blk),
        out_shape=jax.ShapeDtypeStruct((NPAD, D), jnp.float32),
    )(x_p, W1, degp3)

    accp1 = scatter_kernel(g1, idx_both)

    g2 = pl.pallas_call(
        _tc2_body,
        grid=(NBLK,),
        in_specs=[
            pl.BlockSpec((NC, BN, D), lambda i: (0, i, 0)),
            pl.BlockSpec((BN, D), blk),
            pl.BlockSpec((NC, BN, 1), lambda i: (0, i, 0)),
            pl.BlockSpec((D, D), lambda i: (0, 0)),
            pl.BlockSpec((1, D), lambda i: (0, 0)),
        ],
        out_specs=pl.BlockSpec((BN, D), blk),
        out_shape=jax.ShapeDtypeStruct((NPAD, D), jnp.float32),
    )(accp1, g1, degp3, W2, b1r)

    accp2 = scatter_kernel(g2, idx_both)

    pooled = pl.pallas_call(
        _tc3_body,
        grid=(NBLK,),
        in_specs=[
            pl.BlockSpec((NC, BN, D), lambda i: (0, i, 0)),
            pl.BlockSpec((BN, D), blk),
            pl.BlockSpec((NC, BN, 1), lambda i: (0, i, 0)),
            pl.BlockSpec((1, D), lambda i: (0, 0)),
            pl.BlockSpec((BN, 1), blk),
            pl.BlockSpec((BN, 1), blk),
        ],
        out_specs=pl.BlockSpec((G, D), lambda i: (0, 0)),
        out_shape=jax.ShapeDtypeStruct((G, D), jnp.float32),
    )(accp2, g2, degp3, b2r, isn_p, batch_p)

    return pooled


# speedup vs baseline: 2.8377x; 2.8377x over previous
import functools

import jax
import jax.numpy as jnp
from jax import lax
from jax.experimental import pallas as pl
from jax.experimental.pallas import tpu as pltpu
from jax.experimental.pallas import tpu_sc as plsc

N = 10000
E = 320000
D = 128
G = 64

NC = 2
NS = 16
NW = NC * NS

NPAD = 10240
TRASH = N
CH = 80
B = 128
EPAD = NW * CH * B

CHT = 160
CH0 = 80

BN = 2048
NBLK = NPAD // BN


def _make_deg_kernel():
    mesh = plsc.VectorSubcoreMesh(core_axis_name="c", subcore_axis_name="s")

    @functools.partial(
        pl.kernel,
        out_type=jax.ShapeDtypeStruct((NC, NPAD), jnp.float32),
        mesh=mesh,
        scratch_types=[
            pltpu.VMEM((CH, B), jnp.int32),
            pltpu.VMEM((B,), jnp.float32),
            pltpu.VMEM((640,), jnp.float32),
            pltpu.VMEM_SHARED((NPAD,), jnp.float32),
        ],
    )
    def deg_kernel(dst_hbm, degp_hbm, didx, ones_v, zb, degS):
        c = lax.axis_index("c")
        s = lax.axis_index("s")
        wid = s * NC + c

        def zb_body(i, carry):
            zb[pl.ds(i * 16, 16)] = jnp.zeros((16,), jnp.float32)
            return carry
        lax.fori_loop(0, 640 // 16, zb_body, 0)
        for k in range(B // 16):
            ones_v[pl.ds(k * 16, 16)] = jnp.ones((16,), jnp.float32)

        pltpu.sync_copy(zb, degS.at[pl.ds(s * 640, 640)])
        plsc.subcore_barrier()

        pltpu.sync_copy(dst_hbm.at[wid], didx)

        def body(j, carry):
            pltpu.sync_copy(ones_v, degS.at[didx.at[j]], add=True)
            return carry
        lax.fori_loop(0, CH, body, 0)

        plsc.subcore_barrier()
        pltpu.sync_copy(degS.at[pl.ds(s * 640, 640)],
                        degp_hbm.at[c, pl.ds(s * 640, 640)])

    return deg_kernel


def _make_scatter_kernel():
    mesh = plsc.VectorSubcoreMesh(core_axis_name="c", subcore_axis_name="s")

    @functools.partial(
        pl.kernel,
        out_type=jax.ShapeDtypeStruct((NC, NPAD, D), jnp.float32),
        mesh=mesh,
        scratch_types=[
            pltpu.VMEM((4, 2, B), jnp.int32),
            pltpu.VMEM((2, B, D), jnp.float32),
            pltpu.VMEM_SHARED((NPAD, D), jnp.float32),
            pltpu.SemaphoreType.DMA,
            pltpu.SemaphoreType.DMA,
            pltpu.SemaphoreType.DMA,
            pltpu.SemaphoreType.DMA,
            pltpu.SemaphoreType.DMA,
            pltpu.SemaphoreType.DMA,
        ],
    )
    def scatter_kernel(g_hbm, idx_hbm, out_hbm, ibuf, rbuf, accS,
                       is0, is1, is2, is3, gs0, gs1):
        c = lax.axis_index("c")
        s = lax.axis_index("s")
        isems = (is0, is1, is2, is3)
        gsems = (gs0, gs1)

        def zr(r, carry):
            for k in range(D // 16):
                rbuf[0, r, pl.ds(k * 16, 16)] = jnp.zeros((16,), jnp.float32)
            return carry
        lax.fori_loop(0, B, zr, 0)
        for k in range(640 // B):
            pltpu.sync_copy(rbuf.at[0], accS.at[pl.ds(s * 640 + k * B, B)])
        plsc.subcore_barrier()

        def sweep(base, nch):
            pltpu.async_copy(idx_hbm.at[s, base], ibuf.at[0], isems[0])
            pltpu.async_copy(idx_hbm.at[s, base + 1], ibuf.at[1], isems[1])
            pltpu.make_async_copy(idx_hbm.at[s, base], ibuf.at[0],
                                  isems[0]).wait()
            pltpu.async_copy(g_hbm.at[ibuf.at[0, 0]], rbuf.at[0], gsems[0])

            def body(q, carry):
                for k in range(4):
                    l = 4 * q + k
                    j = base + l
                    k1 = (k + 1) % 4
                    k2 = (k + 2) % 4

                    @pl.when(l + 2 < nch)
                    def _():
                        pltpu.async_copy(idx_hbm.at[s, j + 2], ibuf.at[k2],
                                         isems[k2])

                    @pl.when(l + 1 < nch)
                    def _():
                        pltpu.make_async_copy(idx_hbm.at[s, j + 1],
                                              ibuf.at[k1], isems[k1]).wait()
                        pltpu.async_copy(g_hbm.at[ibuf.at[k1, 0]],
                                         rbuf.at[(k + 1) % 2],
                                         gsems[(k + 1) % 2])

                    pltpu.make_async_copy(g_hbm.at[ibuf.at[k, 0]],
                                          rbuf.at[k % 2], gsems[k % 2]).wait()
                    pltpu.sync_copy(rbuf.at[k % 2], accS.at[ibuf.at[k, 1]],
                                    add=True)
                return carry
            lax.fori_loop(0, nch // 4, body, 0)

        @pl.when(c == 0)
        def _():
            sweep(0, CH0)

        @pl.when(c == 1)
        def _():
            sweep(CH0, CHT - CH0)

        plsc.subcore_barrier()
        for k in range(640 // B):
            pltpu.sync_copy(accS.at[pl.ds(s * 640 + k * B, B)],
                            out_hbm.at[c, pl.ds(s * 640 + k * B, B)])

    return scatter_kernel


def _tc1_body(x_ref, w_ref, degp_ref, g_ref):
    deg = degp_ref[0] + degp_ref[1] + 1.0
    dis = lax.rsqrt(deg)
    h = jnp.dot(x_ref[...], w_ref[...], preferred_element_type=jnp.float32)
    g_ref[...] = h * dis


def _tc2_body(accp_ref, g1_ref, degp_ref, w_ref, b_ref, g2_ref):
    deg = degp_ref[0] + degp_ref[1] + 1.0
    dis = lax.rsqrt(deg)
    ssum = accp_ref[0] + accp_ref[1] + g1_ref[...]
    h1 = jnp.maximum(ssum * dis + b_ref[...], 0.0)
    g2_ref[...] = jnp.dot(h1, w_ref[...],
                          preferred_element_type=jnp.float32) * dis


def _tc3_body(accp_ref, g2_ref, degp_ref, b_ref, isn_ref, batch_ref, out_ref):
    i = pl.program_id(0)
    deg = degp_ref[0] + degp_ref[1] + 1.0
    dis = lax.rsqrt(deg)
    h2 = (accp_ref[0] + accp_ref[1] + g2_ref[...]) * dis + b_ref[...]
    h2 = h2 * (0.5 * isn_ref[...])
    onehot = (lax.broadcasted_iota(jnp.int32, (BN, G), 1)
              == batch_ref[...]).astype(jnp.float32)
    contrib = lax.dot_general(onehot, h2, (((0,), (0,)), ((), ())),
                              preferred_element_type=jnp.float32)

    @pl.when(i == 0)
    def _():
        out_ref[...] = contrib

    @pl.when(i != 0)
    def _():
        out_ref[...] = out_ref[...] + contrib


def kernel(x, edge_index, is_neighbor, batch, W1, b1, W2, b2):
    src = edge_index[0]
    dst = edge_index[1]
    npad_e = EPAD - E
    pad_i = jnp.arange(npad_e, dtype=jnp.int32)
    src_pad = jnp.concatenate([src, pad_i % N])
    dst_pad = jnp.concatenate([dst, N + pad_i % (NPAD - N)])
    dst_r = dst_pad.reshape(NW, CH, B)
    idx_both = jnp.stack(
        [src_pad.reshape(NS, CHT, B), dst_pad.reshape(NS, CHT, B)], axis=2)

    x_p = jnp.pad(x, ((0, NPAD - N), (0, 0)))
    isn_p = jnp.pad(is_neighbor, (0, NPAD - N)).reshape(NPAD, 1)
    batch_p = jnp.pad(batch, (0, NPAD - N),
                      constant_values=G).reshape(NPAD, 1)
    b1r = b1.reshape(1, D)
    b2r = b2.reshape(1, D)

    deg_kernel = _make_deg_kernel()
    scatter_kernel = _make_scatter_kernel()

    degp = deg_kernel(dst_r)
    degp3 = degp.reshape(NC, NPAD, 1)

    blk = lambda i: (i, 0)
    g1 = pl.pallas_call(
        _tc1_body,
        grid=(NBLK,),
        in_specs=[
            pl.BlockSpec((BN, D), blk),
            pl.BlockSpec((D, D), lambda i: (0, 0)),
            pl.BlockSpec((NC, BN, 1), lambda i: (0, i, 0)),
        ],
        out_specs=pl.BlockSpec((BN, D), blk),
        out_shape=jax.ShapeDtypeStruct((NPAD, D), jnp.float32),
    )(x_p, W1, degp3)

    accp1 = scatter_kernel(g1, idx_both)

    g2 = pl.pallas_call(
        _tc2_body,
        grid=(NBLK,),
        in_specs=[
            pl.BlockSpec((NC, BN, D), lambda i: (0, i, 0)),
            pl.BlockSpec((BN, D), blk),
            pl.BlockSpec((NC, BN, 1), lambda i: (0, i, 0)),
            pl.BlockSpec((D, D), lambda i: (0, 0)),
            pl.BlockSpec((1, D), lambda i: (0, 0)),
        ],
        out_specs=pl.BlockSpec((BN, D), blk),
        out_shape=jax.ShapeDtypeStruct((NPAD, D), jnp.float32),
    )(accp1, g1, degp3, W2, b1r)

    accp2 = scatter_kernel(g2, idx_both)

    pooled = pl.pallas_call(
        _tc3_body,
        grid=(NBLK,),
        in_specs=[
            pl.BlockSpec((NC, BN, D), lambda i: (0, i, 0)),
            pl.BlockSpec((BN, D), blk),
            pl.BlockSpec((NC, BN, 1), lambda i: (0, i, 0)),
            pl.BlockSpec((1, D), lambda i: (0, 0)),
            pl.BlockSpec((BN, 1), blk),
            pl.BlockSpec((BN, 1), blk),
        ],
        out_specs=pl.BlockSpec((G, D), lambda i: (0, 0)),
        out_shape=jax.ShapeDtypeStruct((G, D), jnp.float32),
    )(accp2, g2, degp3, b2r, isn_p, batch_p)

    return pooled


# speedup vs baseline: 2.9089x; 1.0251x over previous
import functools

import jax
import jax.numpy as jnp
from jax import lax
from jax.experimental import pallas as pl
from jax.experimental.pallas import tpu as pltpu
from jax.experimental.pallas import tpu_sc as plsc

N = 10000
E = 320000
D = 128
G = 64

NC = 2
NS = 16
NW = NC * NS

NPAD = 10240
TRASH = N
CH = 80
B = 128
EPAD = NW * CH * B

CHT = 160
CH0 = 80

BN = 2000
NBLK = N // BN


def _make_deg_kernel():
    mesh = plsc.VectorSubcoreMesh(core_axis_name="c", subcore_axis_name="s")

    @functools.partial(
        pl.kernel,
        out_type=jax.ShapeDtypeStruct((NC, NPAD), jnp.float32),
        mesh=mesh,
        scratch_types=[
            pltpu.VMEM((2 * CH, B), jnp.int32),
            pltpu.VMEM((B,), jnp.float32),
            pltpu.VMEM((640,), jnp.float32),
            pltpu.VMEM_SHARED((NPAD,), jnp.float32),
        ],
    )
    def deg_kernel(idx_hbm, degp_hbm, didx, ones_v, zb, degS):
        c = lax.axis_index("c")
        s = lax.axis_index("s")
        wid = s * NC + c

        def zb_body(i, carry):
            zb[pl.ds(i * 16, 16)] = jnp.zeros((16,), jnp.float32)
            return carry
        lax.fori_loop(0, 640 // 16, zb_body, 0)
        for k in range(B // 16):
            ones_v[pl.ds(k * 16, 16)] = jnp.ones((16,), jnp.float32)

        pltpu.sync_copy(zb, degS.at[pl.ds(s * 640, 640)])
        plsc.subcore_barrier()

        pltpu.sync_copy(idx_hbm.at[wid], didx)

        def body(j, carry):
            pltpu.sync_copy(ones_v, degS.at[didx.at[2 * j + 1]], add=True)
            return carry
        lax.fori_loop(0, CH, body, 0)

        plsc.subcore_barrier()
        pltpu.sync_copy(degS.at[pl.ds(s * 640, 640)],
                        degp_hbm.at[c, pl.ds(s * 640, 640)])

    return deg_kernel


def _make_scatter_kernel():
    mesh = plsc.VectorSubcoreMesh(core_axis_name="c", subcore_axis_name="s")

    @functools.partial(
        pl.kernel,
        out_type=jax.ShapeDtypeStruct((NC, NPAD, D), jnp.float32),
        mesh=mesh,
        scratch_types=[
            pltpu.VMEM((4, 2, B), jnp.int32),
            pltpu.VMEM((2, B, D), jnp.float32),
            pltpu.VMEM_SHARED((NPAD, D), jnp.float32),
            pltpu.SemaphoreType.DMA,
            pltpu.SemaphoreType.DMA,
            pltpu.SemaphoreType.DMA,
            pltpu.SemaphoreType.DMA,
            pltpu.SemaphoreType.DMA,
            pltpu.SemaphoreType.DMA,
        ],
    )
    def scatter_kernel(g_hbm, idx_hbm, out_hbm, ibuf, rbuf, accS,
                       is0, is1, is2, is3, gs0, gs1):
        c = lax.axis_index("c")
        s = lax.axis_index("s")
        isems = (is0, is1, is2, is3)
        gsems = (gs0, gs1)

        def zr(r, carry):
            for k in range(D // 16):
                rbuf[0, r, pl.ds(k * 16, 16)] = jnp.zeros((16,), jnp.float32)
            return carry
        lax.fori_loop(0, B, zr, 0)
        for k in range(640 // B):
            pltpu.sync_copy(rbuf.at[0], accS.at[pl.ds(s * 640 + k * B, B)])
        plsc.subcore_barrier()

        def sweep(base, nch):
            pltpu.async_copy(idx_hbm.at[s, base], ibuf.at[0], isems[0])
            pltpu.async_copy(idx_hbm.at[s, base + 1], ibuf.at[1], isems[1])
            pltpu.make_async_copy(idx_hbm.at[s, base], ibuf.at[0],
                                  isems[0]).wait()
            pltpu.async_copy(g_hbm.at[ibuf.at[0, 0]], rbuf.at[0], gsems[0])

            def body(q, carry):
                for k in range(4):
                    l = 4 * q + k
                    j = base + l
                    k1 = (k + 1) % 4
                    k2 = (k + 2) % 4

                    @pl.when(l + 2 < nch)
                    def _():
                        pltpu.async_copy(idx_hbm.at[s, j + 2], ibuf.at[k2],
                                         isems[k2])

                    @pl.when(l + 1 < nch)
                    def _():
                        pltpu.make_async_copy(idx_hbm.at[s, j + 1],
                                              ibuf.at[k1], isems[k1]).wait()
                        pltpu.async_copy(g_hbm.at[ibuf.at[k1, 0]],
                                         rbuf.at[(k + 1) % 2],
                                         gsems[(k + 1) % 2])

                    pltpu.make_async_copy(g_hbm.at[ibuf.at[k, 0]],
                                          rbuf.at[k % 2], gsems[k % 2]).wait()
                    pltpu.sync_copy(rbuf.at[k % 2], accS.at[ibuf.at[k, 1]],
                                    add=True)
                return carry
            lax.fori_loop(0, nch // 4, body, 0)

        @pl.when(c == 0)
        def _():
            sweep(0, CH0)

        @pl.when(c == 1)
        def _():
            sweep(CH0, CHT - CH0)

        plsc.subcore_barrier()
        for k in range(640 // B):
            pltpu.sync_copy(accS.at[pl.ds(s * 640 + k * B, B)],
                            out_hbm.at[c, pl.ds(s * 640 + k * B, B)])

    return scatter_kernel


def _mm_body(x_ref, w_ref, u_ref):
    u_ref[...] = jnp.dot(x_ref[...], w_ref[...],
                         preferred_element_type=jnp.float32)


def _scale_body(u_ref, degp_ref, g_ref):
    deg = degp_ref[0] + degp_ref[1] + 1.0
    g_ref[...] = u_ref[...] * lax.rsqrt(deg)


def _tc2_body(accp_ref, g1_ref, degp_ref, w_ref, b_ref, g2_ref):
    deg = degp_ref[0] + degp_ref[1] + 1.0
    dis = lax.rsqrt(deg)
    ssum = accp_ref[0] + accp_ref[1] + g1_ref[...]
    h1 = jnp.maximum(ssum * dis + b_ref[...], 0.0)
    g2_ref[...] = jnp.dot(h1, w_ref[...],
                          preferred_element_type=jnp.float32) * dis


def _tc3_body(accp_ref, g2_ref, degp_ref, b_ref, isn_ref, batch_ref, out_ref):
    i = pl.program_id(0)
    deg = degp_ref[0] + degp_ref[1] + 1.0
    dis = lax.rsqrt(deg)
    h2 = (accp_ref[0] + accp_ref[1] + g2_ref[...]) * dis + b_ref[...]
    h2 = h2 * (0.5 * isn_ref[...])
    onehot = (lax.broadcasted_iota(jnp.int32, (BN, G), 1)
              == batch_ref[...]).astype(jnp.float32)
    contrib = lax.dot_general(onehot, h2, (((0,), (0,)), ((), ())),
                              preferred_element_type=jnp.float32)

    @pl.when(i == 0)
    def _():
        out_ref[...] = contrib

    @pl.when(i != 0)
    def _():
        out_ref[...] = out_ref[...] + contrib


def kernel(x, edge_index, is_neighbor, batch, W1, b1, W2, b2):
    npad_e = EPAD - E
    pad_i = jnp.arange(npad_e, dtype=jnp.int32)
    pad_arr = jnp.stack([pad_i % N, N + pad_i % (NPAD - N)])
    ei_pad = jnp.concatenate([edge_index, pad_arr], axis=1)
    idx_both = ei_pad.reshape(2, NW * CH, B).transpose(1, 0, 2).reshape(
        NW, CH, 2, B)
    idx_flat = idx_both.reshape(NW, 2 * CH, B)

    isn_p = is_neighbor.reshape(N, 1)
    batch_p = batch.reshape(N, 1)
    b1r = b1.reshape(1, D)
    b2r = b2.reshape(1, D)

    deg_kernel = _make_deg_kernel()
    scatter_kernel = _make_scatter_kernel()

    degp = deg_kernel(idx_flat)
    degp3 = degp.reshape(NC, NPAD, 1)

    blk = lambda i: (i, 0)
    u1 = pl.pallas_call(
        _mm_body,
        grid=(NBLK,),
        in_specs=[
            pl.BlockSpec((BN, D), blk),
            pl.BlockSpec((D, D), lambda i: (0, 0)),
        ],
        out_specs=pl.BlockSpec((BN, D), blk),
        out_shape=jax.ShapeDtypeStruct((N, D), jnp.float32),
    )(x, W1)

    g1 = pl.pallas_call(
        _scale_body,
        grid=(NBLK,),
        in_specs=[
            pl.BlockSpec((BN, D), blk),
            pl.BlockSpec((NC, BN, 1), lambda i: (0, i, 0)),
        ],
        out_specs=pl.BlockSpec((BN, D), blk),
        out_shape=jax.ShapeDtypeStruct((N, D), jnp.float32),
    )(u1, degp3)

    accp1 = scatter_kernel(g1, idx_both)

    g2 = pl.pallas_call(
        _tc2_body,
        grid=(NBLK,),
        in_specs=[
            pl.BlockSpec((NC, BN, D), lambda i: (0, i, 0)),
            pl.BlockSpec((BN, D), blk),
            pl.BlockSpec((NC, BN, 1), lambda i: (0, i, 0)),
            pl.BlockSpec((D, D), lambda i: (0, 0)),
            pl.BlockSpec((1, D), lambda i: (0, 0)),
        ],
        out_specs=pl.BlockSpec((BN, D), blk),
        out_shape=jax.ShapeDtypeStruct((N, D), jnp.float32),
    )(accp1, g1, degp3, W2, b1r)

    accp2 = scatter_kernel(g2, idx_both)

    pooled = pl.pallas_call(
        _tc3_body,
        grid=(NBLK,),
        in_specs=[
            pl.BlockSpec((NC, BN, D), lambda i: (0, i, 0)),
            pl.BlockSpec((BN, D), blk),
            pl.BlockSpec((NC, BN, 1), lambda i: (0, i, 0)),
            pl.BlockSpec((1, D), lambda i: (0, 0)),
            pl.BlockSpec((BN, 1), blk),
            pl.BlockSpec((BN, 1), blk),
        ],
        out_specs=pl.BlockSpec((G, D), lambda i: (0, 0)),
        out_shape=jax.ShapeDtypeStruct((G, D), jnp.float32),
    )(accp2, g2, degp3, b2r, isn_p, batch_p)

    return pooled


# speedup vs baseline: 2.9294x; 1.0070x over previous
import functools

import jax
import jax.numpy as jnp
from jax import lax
from jax.experimental import pallas as pl
from jax.experimental.pallas import tpu as pltpu
from jax.experimental.pallas import tpu_sc as plsc

N = 10000
E = 320000
D = 128
G = 64

NC = 2
NS = 16
NW = NC * NS

NPAD = 10240
TRASH = N
CH = 80
B = 128
EPAD = NW * CH * B

CHT = 160
CH0 = 80

BN = 2000
NBLK = N // BN


def _make_deg_kernel():
    mesh = plsc.VectorSubcoreMesh(core_axis_name="c", subcore_axis_name="s")

    @functools.partial(
        pl.kernel,
        out_type=jax.ShapeDtypeStruct((NC, NPAD), jnp.float32),
        mesh=mesh,
        scratch_types=[
            pltpu.VMEM((2 * CH, B), jnp.int32),
            pltpu.VMEM((B,), jnp.float32),
            pltpu.VMEM((640,), jnp.float32),
            pltpu.VMEM_SHARED((NPAD,), jnp.float32),
        ],
    )
    def deg_kernel(idx_hbm, degp_hbm, didx, ones_v, zb, degS):
        c = lax.axis_index("c")
        s = lax.axis_index("s")
        wid = s * NC + c

        def zb_body(i, carry):
            zb[pl.ds(i * 16, 16)] = jnp.zeros((16,), jnp.float32)
            return carry
        lax.fori_loop(0, 640 // 16, zb_body, 0)
        for k in range(B // 16):
            ones_v[pl.ds(k * 16, 16)] = jnp.ones((16,), jnp.float32)

        pltpu.sync_copy(zb, degS.at[pl.ds(s * 640, 640)])
        plsc.subcore_barrier()

        pltpu.sync_copy(idx_hbm.at[wid], didx)

        def body(j, carry):
            pltpu.sync_copy(ones_v, degS.at[didx.at[2 * j + 1]], add=True)
            return carry
        lax.fori_loop(0, CH, body, 0)

        plsc.subcore_barrier()
        pltpu.sync_copy(degS.at[pl.ds(s * 640, 640)],
                        degp_hbm.at[c, pl.ds(s * 640, 640)])

    return deg_kernel


def _make_scatter_kernel():
    mesh = plsc.VectorSubcoreMesh(core_axis_name="c", subcore_axis_name="s")

    @functools.partial(
        pl.kernel,
        out_type=jax.ShapeDtypeStruct((NC, NPAD, D), jnp.float32),
        mesh=mesh,
        scratch_types=[
            pltpu.VMEM((4, 2, B), jnp.int32),
            pltpu.VMEM((2, B, D), jnp.float32),
            pltpu.VMEM_SHARED((NPAD, D), jnp.float32),
            pltpu.SemaphoreType.DMA,
            pltpu.SemaphoreType.DMA,
            pltpu.SemaphoreType.DMA,
            pltpu.SemaphoreType.DMA,
            pltpu.SemaphoreType.DMA,
            pltpu.SemaphoreType.DMA,
        ],
    )
    def scatter_kernel(g_hbm, idx_hbm, out_hbm, ibuf, rbuf, accS,
                       is0, is1, is2, is3, gs0, gs1):
        c = lax.axis_index("c")
        s = lax.axis_index("s")
        isems = (is0, is1, is2, is3)
        gsems = (gs0, gs1)

        def zr(r, carry):
            for k in range(D // 16):
                rbuf[0, r, pl.ds(k * 16, 16)] = jnp.zeros((16,), jnp.float32)
            return carry
        lax.fori_loop(0, B, zr, 0)
        for k in range(640 // B):
            pltpu.sync_copy(rbuf.at[0], accS.at[pl.ds(s * 640 + k * B, B)])
        plsc.subcore_barrier()

        def sweep(base, nch):
            pltpu.async_copy(idx_hbm.at[s, base], ibuf.at[0], isems[0])
            pltpu.async_copy(idx_hbm.at[s, base + 1], ibuf.at[1], isems[1])
            pltpu.make_async_copy(idx_hbm.at[s, base], ibuf.at[0],
                                  isems[0]).wait()
            pltpu.async_copy(g_hbm.at[ibuf.at[0, 0]], rbuf.at[0], gsems[0])

            def body(q, carry):
                for k in range(4):
                    l = 4 * q + k
                    j = base + l
                    k1 = (k + 1) % 4
                    k2 = (k + 2) % 4

                    @pl.when(l + 2 < nch)
                    def _():
                        pltpu.async_copy(idx_hbm.at[s, j + 2], ibuf.at[k2],
                                         isems[k2])

                    @pl.when(l + 1 < nch)
                    def _():
                        pltpu.make_async_copy(idx_hbm.at[s, j + 1],
                                              ibuf.at[k1], isems[k1]).wait()
                        pltpu.async_copy(g_hbm.at[ibuf.at[k1, 0]],
                                         rbuf.at[(k + 1) % 2],
                                         gsems[(k + 1) % 2])

                    pltpu.make_async_copy(g_hbm.at[ibuf.at[k, 0]],
                                          rbuf.at[k % 2], gsems[k % 2]).wait()
                    pltpu.sync_copy(rbuf.at[k % 2], accS.at[ibuf.at[k, 1]],
                                    add=True)
                return carry
            lax.fori_loop(0, nch // 4, body, 0)

        @pl.when(c == 0)
        def _():
            sweep(0, CH0)

        @pl.when(c == 1)
        def _():
            sweep(CH0, CHT - CH0)

        plsc.subcore_barrier()
        for k in range(640 // B):
            pltpu.sync_copy(accS.at[pl.ds(s * 640 + k * B, B)],
                            out_hbm.at[c, pl.ds(s * 640 + k * B, B)])

    return scatter_kernel


def _mm_body(x_ref, w_ref, u_ref):
    u_ref[...] = jnp.dot(x_ref[...], w_ref[...],
                         preferred_element_type=jnp.float32)


def _scale_body(u_ref, degp_ref, g_ref):
    deg = degp_ref[0] + degp_ref[1] + 1.0
    g_ref[...] = u_ref[...] * lax.rsqrt(deg)


def _tc2_body(accp_ref, g1_ref, degp_ref, w_ref, b_ref, g2_ref):
    deg = degp_ref[0] + degp_ref[1] + 1.0
    dis = lax.rsqrt(deg)
    ssum = accp_ref[0] + accp_ref[1] + g1_ref[...]
    h1 = jnp.maximum(ssum * dis + b_ref[...], 0.0)
    g2_ref[...] = jnp.dot(h1, w_ref[...],
                          preferred_element_type=jnp.float32) * dis


def _tc3_body(accp_ref, g2_ref, degp_ref, b_ref, isn_ref, batch_ref, out_ref):
    i = pl.program_id(0)
    deg = degp_ref[0] + degp_ref[1] + 1.0
    dis = lax.rsqrt(deg)
    h2 = (accp_ref[0] + accp_ref[1] + g2_ref[...]) * dis + b_ref[...]
    h2 = h2 * (0.5 * isn_ref[...])
    onehot = (lax.broadcasted_iota(jnp.int32, (BN, G), 1)
              == batch_ref[...]).astype(jnp.float32)
    contrib = lax.dot_general(onehot, h2, (((0,), (0,)), ((), ())),
                              preferred_element_type=jnp.float32)

    @pl.when(i == 0)
    def _():
        out_ref[...] = contrib

    @pl.when(i != 0)
    def _():
        out_ref[...] = out_ref[...] + contrib


def kernel(x, edge_index, is_neighbor, batch, W1, b1, W2, b2):
    npad_e = EPAD - E
    pad_i = jnp.arange(npad_e, dtype=jnp.int32)
    pad_arr = jnp.stack([pad_i % N, N + pad_i % (NPAD - N)])
    ei_pad = jnp.concatenate([edge_index, pad_arr], axis=1)
    idx_both = ei_pad.reshape(2, NW * CH, B).transpose(1, 0, 2).reshape(
        NS, CHT, 2, B)
    idx_flat = idx_both.reshape(NW, 2 * CH, B)

    isn_p = is_neighbor.reshape(N, 1)
    batch_p = batch.reshape(N, 1)
    b1r = b1.reshape(1, D)
    b2r = b2.reshape(1, D)

    deg_kernel = _make_deg_kernel()
    scatter_kernel = _make_scatter_kernel()

    degp = deg_kernel(idx_flat)
    degp3 = degp.reshape(NC, NPAD, 1)

    blk = lambda i: (i, 0)
    u1 = pl.pallas_call(
        _mm_body,
        grid=(NBLK,),
        in_specs=[
            pl.BlockSpec((BN, D), blk),
            pl.BlockSpec((D, D), lambda i: (0, 0)),
        ],
        out_specs=pl.BlockSpec((BN, D), blk),
        out_shape=jax.ShapeDtypeStruct((N, D), jnp.float32),
    )(x, W1)

    g1 = pl.pallas_call(
        _scale_body,
        grid=(NBLK,),
        in_specs=[
            pl.BlockSpec((BN, D), blk),
            pl.BlockSpec((NC, BN, 1), lambda i: (0, i, 0)),
        ],
        out_specs=pl.BlockSpec((BN, D), blk),
        out_shape=jax.ShapeDtypeStruct((N, D), jnp.float32),
    )(u1, degp3)

    accp1 = scatter_kernel(g1, idx_both)

    g2 = pl.pallas_call(
        _tc2_body,
        grid=(NBLK,),
        in_specs=[
            pl.BlockSpec((NC, BN, D), lambda i: (0, i, 0)),
            pl.BlockSpec((BN, D), blk),
            pl.BlockSpec((NC, BN, 1), lambda i: (0, i, 0)),
            pl.BlockSpec((D, D), lambda i: (0, 0)),
            pl.BlockSpec((1, D), lambda i: (0, 0)),
        ],
        out_specs=pl.BlockSpec((BN, D), blk),
        out_shape=jax.ShapeDtypeStruct((N, D), jnp.float32),
    )(accp1, g1, degp3, W2, b1r)

    accp2 = scatter_kernel(g2, idx_both)

    pooled = pl.pallas_call(
        _tc3_body,
        grid=(NBLK,),
        in_specs=[
            pl.BlockSpec((NC, BN, D), lambda i: (0, i, 0)),
            pl.BlockSpec((BN, D), blk),
            pl.BlockSpec((NC, BN, 1), lambda i: (0, i, 0)),
            pl.BlockSpec((1, D), lambda i: (0, 0)),
            pl.BlockSpec((BN, 1), blk),
            pl.BlockSpec((BN, 1), blk),
        ],
        out_specs=pl.BlockSpec((G, D), lambda i: (0, 0)),
        out_shape=jax.ShapeDtypeStruct((G, D), jnp.float32),
    )(accp2, g2, degp3, b2r, isn_p, batch_p)

    return pooled
